# Initial kernel scaffold; baseline (speedup 1.0000x reference)
import jax
import jax.numpy as jnp
from jax.experimental import pallas as pl


def kernel(input_ids, table, W, b):
    raise NotImplementedError("write your pallas kernel here")


# speedup vs baseline: 4.8455x; 4.8455x over previous
import functools

import jax
import jax.numpy as jnp
from jax import lax
from jax.experimental import pallas as pl
from jax.experimental.pallas import tpu as pltpu
from jax.experimental.pallas import tpu_sc as plsc

B = 4096
S = 50
D = 128
C = 1000
CP = 1024

NC = 2
NS = 16
NW = NC * NS
BPW = B // NW
L = 16
DL = D // L


def _pool_body(ids_hbm, table_hbm, out_hbm, idx_v, rows_v, pooled_v, sem):
    wid = lax.axis_index("s") * NC + lax.axis_index("c")
    base = wid * BPW
    pltpu.sync_copy(ids_hbm.at[pl.ds(base, BPW), :], idx_v)

    def elem(e, carry):
        pltpu.async_copy(table_hbm.at[idx_v.at[e]], rows_v, sem).wait()

        def row(r, accs):
            return tuple(accs[d] + rows_v[r, pl.ds(d * L, L)] for d in range(DL))

        accs = lax.fori_loop(
            0, S, row, tuple(jnp.zeros((L,), jnp.float32) for _ in range(DL))
        )
        for d in range(DL):
            pooled_v[e, pl.ds(d * L, L)] = accs[d]
        return carry

    lax.fori_loop(0, BPW, elem, 0)
    pltpu.sync_copy(pooled_v, out_hbm.at[pl.ds(base, BPW), :])


def _pool(ids, table):
    mesh = plsc.VectorSubcoreMesh(core_axis_name="c", subcore_axis_name="s")
    return pl.kernel(
        _pool_body,
        out_type=jax.ShapeDtypeStruct((B, D), jnp.float32),
        mesh=mesh,
        scratch_types=[
            pltpu.VMEM((BPW, S), jnp.int32),
            pltpu.VMEM((S, D), jnp.float32),
            pltpu.VMEM((BPW, D), jnp.float32),
            pltpu.SemaphoreType.DMA,
        ],
    )(ids, table)


BT = 512


def _mm_body(p_ref, w_ref, b_ref, o_ref):
    o_ref[...] = (
        jnp.dot(p_ref[...] * (1.0 / S), w_ref[...],
                preferred_element_type=jnp.float32)
        + b_ref[...]
    )


def _matmul(pooled, w_pad, b_pad):
    return pl.pallas_call(
        _mm_body,
        grid=(B // BT,),
        in_specs=[
            pl.BlockSpec((BT, D), lambda i: (i, 0)),
            pl.BlockSpec((D, CP), lambda i: (0, 0)),
            pl.BlockSpec((1, CP), lambda i: (0, 0)),
        ],
        out_specs=pl.BlockSpec((BT, CP), lambda i: (i, 0)),
        out_shape=jax.ShapeDtypeStruct((B, CP), jnp.float32),
    )(pooled, w_pad, b_pad)


@jax.jit
def kernel(input_ids, table, W, b):
    ids = input_ids.astype(jnp.int32)
    pooled = _pool(ids, table)
    w_pad = jnp.pad(W, ((0, 0), (0, CP - C)))
    b_pad = jnp.pad(b, (0, CP - C)).reshape(1, CP)
    logits = _matmul(pooled, w_pad, b_pad)
    return logits[:, :C]


# speedup vs baseline: 9.9926x; 2.0622x over previous
import functools

import jax
import jax.numpy as jnp
from jax import lax
from jax.experimental import pallas as pl
from jax.experimental.pallas import tpu as pltpu
from jax.experimental.pallas import tpu_sc as plsc

B = 4096
S = 50
D = 128
C = 1000
CP = 1024

NC = 2
NS = 16
NW = NC * NS
BPW = B // NW
L = 16
DL = D // L


NB = 4


def _pool_body(ids_hbm, table_hbm, out_hbm, idx_v, rows_v, pooled_v, sems):
    wid = lax.axis_index("s") * NC + lax.axis_index("c")
    base = wid * BPW
    pltpu.sync_copy(ids_hbm.at[pl.ds(base, BPW), :], idx_v.at[pl.ds(0, BPW), :])
    pltpu.sync_copy(ids_hbm.at[pl.ds(base, NB), :], idx_v.at[pl.ds(BPW, NB), :])

    def fire(e, k):
        pltpu.make_async_copy(
            table_hbm.at[idx_v.at[e]], rows_v.at[k], sems.at[k]
        ).start()

    def accum(e, k):
        def row(r, accs):
            return tuple(
                accs[d] + rows_v[k, r, pl.ds(d * L, L)] for d in range(DL)
            )

        accs = lax.fori_loop(
            0, S, row, tuple(jnp.zeros((L,), jnp.float32) for _ in range(DL))
        )
        for d in range(DL):
            pooled_v[e, pl.ds(d * L, L)] = accs[d]

    def wait(k):
        pltpu.make_async_copy(
            table_hbm.at[idx_v.at[0]], rows_v.at[k], sems.at[k]
        ).wait()

    for k in range(NB - 1):
        fire(k, k)

    def group(g, carry):
        e0 = g * NB
        for k in range(NB):
            fire(e0 + k + NB - 1, (k + NB - 1) % NB)
            wait(k)
            accum(e0 + k, k)
        return carry

    lax.fori_loop(0, BPW // NB, group, 0)
    for k in range(NB - 1):
        wait(k)
    pltpu.sync_copy(pooled_v, out_hbm.at[pl.ds(base, BPW), :])


def _pool(ids, table):
    mesh = plsc.VectorSubcoreMesh(core_axis_name="c", subcore_axis_name="s")
    return pl.kernel(
        _pool_body,
        out_type=jax.ShapeDtypeStruct((B, D), jnp.float32),
        mesh=mesh,
        scratch_types=[
            pltpu.VMEM((BPW + NB, S), jnp.int32),
            pltpu.VMEM((NB, S, D), jnp.float32),
            pltpu.VMEM((BPW, D), jnp.float32),
            pltpu.SemaphoreType.DMA((NB,)),
        ],
    )(ids, table)


BT = 512


def _mm_body(p_ref, w_ref, b_ref, o_ref):
    o_ref[...] = (
        jnp.dot(p_ref[...] * (1.0 / S), w_ref[...],
                preferred_element_type=jnp.float32)
        + b_ref[...]
    )


def _matmul(pooled, w_pad, b_pad):
    return pl.pallas_call(
        _mm_body,
        grid=(B // BT,),
        in_specs=[
            pl.BlockSpec((BT, D), lambda i: (i, 0)),
            pl.BlockSpec((D, CP), lambda i: (0, 0)),
            pl.BlockSpec((1, CP), lambda i: (0, 0)),
        ],
        out_specs=pl.BlockSpec((BT, CP), lambda i: (i, 0)),
        out_shape=jax.ShapeDtypeStruct((B, CP), jnp.float32),
    )(pooled, w_pad, b_pad)


@jax.jit
def kernel(input_ids, table, W, b):
    ids = input_ids.astype(jnp.int32)
    pooled = _pool(ids, table)
    w_pad = jnp.pad(W, ((0, 0), (0, CP - C)))
    b_pad = jnp.pad(b, (0, CP - C)).reshape(1, CP)
    logits = _matmul(pooled, w_pad, b_pad)
    return logits[:, :C]
